# baseline (speedup 1.0000x reference)
import jax
import jax.numpy as jnp
import numpy as np
from jax.experimental import pallas as pl

B, N, D = 16, 2048, 3
P = 32
G = N // P
K = 16
H = 128
T = 64
OUT = 256
NH = 4
DH = T // NH


def _fps(xs, g):
    Bv, Nv, _ = xs.shape
    dists = jnp.full((Bv, Nv), 1e10, dtype=xs.dtype)
    last = jnp.zeros((Bv,), dtype=jnp.int32)
    def step(carry, _):
        dists, last = carry
        lp = xs[jnp.arange(Bv), last][:, None, :]
        d = jnp.sum((xs - lp) ** 2, axis=-1)
        dists = jnp.minimum(dists, d)
        nxt = jnp.argmax(dists, axis=-1).astype(jnp.int32)
        return (dists, nxt), last
    _, idxs = jax.lax.scan(step, (dists, last), None, length=g)
    return jnp.transpose(idxs, (1, 0))


def _out_proj_kernel(pooled_ref, w_ref, b_ref, o_ref):
    o_ref[...] = jnp.dot(pooled_ref[...], w_ref[...],
                         preferred_element_type=jnp.float32) + b_ref[...]


def kernel(x, We1, be1, We2, be2, Wp1, bp1, Wp2, bp2, Wq, Wk, Wv, Wo, Wf1, bf1, Wf2, bf2, Wout, bout):
    xs = jax.lax.stop_gradient(x)
    fidx = _fps(xs, G)
    c = jnp.take_along_axis(xs, fidx[:, :, None], axis=1)
    d2 = jnp.sum((c[:, :, None, :] - xs[:, None, :, :]) ** 2, axis=-1)
    nn_idx = jax.lax.top_k(-d2, P)[1]
    pts = jnp.take_along_axis(xs[:, None, :, :], nn_idx[..., None], axis=2)
    p = pts - c[:, :, None, :]
    pd = jnp.sum((p[..., :, None, :] - p[..., None, :, :]) ** 2, axis=-1)
    kidx = jax.lax.top_k(-pd, K)[1]
    nbr = jnp.take_along_axis(p[:, :, None, :, :], kidx[..., None], axis=3)
    pc = jnp.broadcast_to(p[..., :, None, :], nbr.shape)
    edge = jnp.concatenate([pc, nbr - pc], axis=-1)
    h = jax.nn.relu(edge @ We1 + be1)
    h = jnp.max(h, axis=3)
    t = jax.nn.relu(h @ We2 + be2)
    z = jnp.max(t, axis=2)
    pe = jax.nn.relu(c @ Wp1 + bp1) @ Wp2 + bp2
    z = z + pe
    q = (z @ Wq).reshape(B, G, NH, DH).transpose(0, 2, 1, 3)
    kk = (z @ Wk).reshape(B, G, NH, DH).transpose(0, 2, 1, 3)
    v = (z @ Wv).reshape(B, G, NH, DH).transpose(0, 2, 1, 3)
    att = jax.nn.softmax(jnp.einsum('bhqd,bhkd->bhqk', q, kk) / np.sqrt(DH), axis=-1)
    o = jnp.einsum('bhqk,bhkd->bhqd', att, v).transpose(0, 2, 1, 3).reshape(B, G, T) @ Wo
    z = z + o
    z = z + jax.nn.relu(z @ Wf1 + bf1) @ Wf2 + bf2
    pooled = jnp.mean(z, axis=1)
    return pl.pallas_call(
        _out_proj_kernel,
        out_shape=jax.ShapeDtypeStruct((B, OUT), jnp.float32),
    )(pooled, Wout, bout)


# speedup vs baseline: 19.8127x; 19.8127x over previous
import jax
import jax.numpy as jnp
import numpy as np
from jax.experimental import pallas as pl

B, N, D = 16, 2048, 3
P = 32
G = N // P
K = 16
H = 128
T = 64
OUT = 256
NH = 4
DH = T // NH

_BIG = 1e9
_NEG = -1e30


def _group_kernel(xx_ref, xy_ref, xz_ref,
                  cx_ref, cy_ref, cz_ref, px_ref, py_ref, pz_ref):
    f32 = jnp.float32
    xx = xx_ref[...]
    xy = xy_ref[...]
    xz = xz_ref[...]
    iota_n = jax.lax.broadcasted_iota(jnp.int32, (1, N), 1).astype(f32)
    iota_g = jax.lax.broadcasted_iota(jnp.int32, (1, G), 1).astype(f32)
    iota_p = jax.lax.broadcasted_iota(jnp.int32, (1, P), 1).astype(f32)

    def fps_step(g, carry):
        dists, lpx, lpy, lpz, cxa, cya, cza = carry
        rowsel = (iota_g == g.astype(f32)).astype(f32)
        cxa = cxa + lpx * rowsel
        cya = cya + lpy * rowsel
        cza = cza + lpz * rowsel
        dx = xx - lpx
        dy = xy - lpy
        dz = xz - lpz
        d = (dx * dx + dy * dy) + dz * dz
        dists = jnp.minimum(dists, d)
        m = jnp.max(dists, axis=1, keepdims=True)
        cand = dists == m
        idx = jnp.min(jnp.where(cand, iota_n, float(N)), axis=1, keepdims=True)
        oh = iota_n == idx
        lpx = jnp.sum(jnp.where(oh, xx, 0.0), axis=1, keepdims=True)
        lpy = jnp.sum(jnp.where(oh, xy, 0.0), axis=1, keepdims=True)
        lpz = jnp.sum(jnp.where(oh, xz, 0.0), axis=1, keepdims=True)
        return dists, lpx, lpy, lpz, cxa, cya, cza

    init = (jnp.full((B, N), 1e10, f32),
            xx[:, 0:1], xy[:, 0:1], xz[:, 0:1],
            jnp.zeros((B, G), f32), jnp.zeros((B, G), f32), jnp.zeros((B, G), f32))
    _, _, _, _, cxa, cya, cza = jax.lax.fori_loop(0, G, fps_step, init)
    cx_ref[...] = cxa
    cy_ref[...] = cya
    cz_ref[...] = cza

    ax = cxa[:, :, None] - xx[:, None, :]
    ay = cya[:, :, None] - xy[:, None, :]
    az = cza[:, :, None] - xz[:, None, :]
    d2 = ((ax * ax + ay * ay) + az * az).reshape(B * G, N)

    def sel_step(s, carry):
        d2c, pxa, pya, pza = carry
        m = jnp.min(d2c, axis=1, keepdims=True)
        cand = d2c == m
        idx = jnp.min(jnp.where(cand, iota_n, float(N)), axis=1, keepdims=True)
        oh = iota_n == idx
        oh3 = oh.reshape(B, G, N)
        gx = jnp.sum(jnp.where(oh3, xx[:, None, :], 0.0), axis=2,
                     keepdims=True).reshape(B * G, 1)
        gy = jnp.sum(jnp.where(oh3, xy[:, None, :], 0.0), axis=2,
                     keepdims=True).reshape(B * G, 1)
        gz = jnp.sum(jnp.where(oh3, xz[:, None, :], 0.0), axis=2,
                     keepdims=True).reshape(B * G, 1)
        colsel = (iota_p == s.astype(f32)).astype(f32)
        pxa = pxa + gx * colsel
        pya = pya + gy * colsel
        pza = pza + gz * colsel
        d2c = d2c + jnp.where(oh, _BIG, 0.0)
        return d2c, pxa, pya, pza

    z = jnp.zeros((B * G, P), f32)
    _, pxa, pya, pza = jax.lax.fori_loop(0, P, sel_step, (d2, z, z, z))
    px_ref[...] = pxa.reshape(B, G, P) - cxa[:, :, None]
    py_ref[...] = pya.reshape(B, G, P) - cya[:, :, None]
    pz_ref[...] = pza.reshape(B, G, P) - cza[:, :, None]


def _encode_kernel(px_ref, py_ref, pz_ref, cm_ref,
                   We1_ref, be1_ref, We2_ref, be2_ref,
                   Wp1_ref, bp1_ref, Wp2_ref, bp2_ref,
                   Wq_ref, Wk_ref, Wv_ref, Wo_ref,
                   Wf1_ref, bf1_ref, Wf2_ref, bf2_ref,
                   Wout_ref, bout_ref, o_ref):
    f32 = jnp.float32
    px = px_ref[0]
    py = py_ref[0]
    pz = pz_ref[0]

    dx = px[:, :, None] - px[:, None, :]
    dy = py[:, :, None] - py[:, None, :]
    dz = pz[:, :, None] - pz[:, None, :]
    pd = (dx * dx + dy * dy) + dz * dz
    iota_p = jax.lax.broadcasted_iota(jnp.int32, (1, 1, P), 2).astype(f32)
    mask = jnp.zeros((G, P, P), f32)
    for _ in range(K):
        m = jnp.min(pd, axis=2, keepdims=True)
        cand = pd == m
        idx = jnp.min(jnp.where(cand, iota_p, float(P)), axis=2, keepdims=True)
        oh = (iota_p == idx)
        mask = mask + oh.astype(f32)
        pd = pd + jnp.where(oh, _BIG, 0.0)

    We1 = We1_ref[...]
    wd0 = (We1[0:1, :] - We1[3:4, :])[None]
    wd1 = (We1[1:2, :] - We1[4:5, :])[None]
    wd2 = (We1[2:3, :] - We1[5:6, :])[None]
    wn0 = We1[3:4, :][None]
    wn1 = We1[4:5, :][None]
    wn2 = We1[5:6, :][None]
    a = px[:, :, None] * wd0 + py[:, :, None] * wd1 + pz[:, :, None] * wd2
    b = px[:, :, None] * wn0 + py[:, :, None] * wn1 + pz[:, :, None] * wn2
    mx = jnp.full((G, P, H), _NEG, f32)
    for j in range(P):
        sel = mask[:, :, j:j + 1] > 0.5
        bj = b[:, j:j + 1, :]
        mx = jnp.maximum(mx, jnp.where(sel, bj, _NEG))
    h = jax.nn.relu(a + be1_ref[...][None] + mx)

    t = jax.nn.relu(jnp.dot(h.reshape(G * P, H), We2_ref[...],
                            preferred_element_type=f32) + be2_ref[...])
    z = jnp.max(t.reshape(G, P, T), axis=1)

    cm = cm_ref[0]
    pe = jnp.dot(jax.nn.relu(jnp.dot(cm, Wp1_ref[...],
                                     preferred_element_type=f32) + bp1_ref[...]),
                 Wp2_ref[...], preferred_element_type=f32) + bp2_ref[...]
    z = z + pe

    q = jnp.dot(z, Wq_ref[...], preferred_element_type=f32)
    k = jnp.dot(z, Wk_ref[...], preferred_element_type=f32)
    v = jnp.dot(z, Wv_ref[...], preferred_element_type=f32)
    scale = 1.0 / float(np.sqrt(DH))
    outs = []
    for hh in range(NH):
        sl = slice(hh * DH, (hh + 1) * DH)
        qh, kh, vh = q[:, sl], k[:, sl], v[:, sl]
        s = jax.lax.dot_general(qh, kh, (((1,), (1,)), ((), ())),
                                preferred_element_type=f32) * scale
        s = s - jnp.max(s, axis=1, keepdims=True)
        e = jnp.exp(s)
        att = e / jnp.sum(e, axis=1, keepdims=True)
        outs.append(jnp.dot(att, vh, preferred_element_type=f32))
    o = jnp.concatenate(outs, axis=1)
    z = z + jnp.dot(o, Wo_ref[...], preferred_element_type=f32)

    f = jax.nn.relu(jnp.dot(z, Wf1_ref[...], preferred_element_type=f32)
                    + bf1_ref[...])
    z = z + jnp.dot(f, Wf2_ref[...], preferred_element_type=f32) + bf2_ref[...]
    pooled = jnp.mean(z, axis=0, keepdims=True)
    o_ref[...] = (jnp.dot(pooled, Wout_ref[...],
                          preferred_element_type=f32) + bout_ref[...])[None]


def kernel(x, We1, be1, We2, be2, Wp1, bp1, Wp2, bp2, Wq, Wk, Wv, Wo,
           Wf1, bf1, Wf2, bf2, Wout, bout):
    f32 = jnp.float32
    xx = x[:, :, 0]
    xy = x[:, :, 1]
    xz = x[:, :, 2]

    cx, cy, cz, px, py, pz = pl.pallas_call(
        _group_kernel,
        out_shape=[
            jax.ShapeDtypeStruct((B, G), f32),
            jax.ShapeDtypeStruct((B, G), f32),
            jax.ShapeDtypeStruct((B, G), f32),
            jax.ShapeDtypeStruct((B, G, P), f32),
            jax.ShapeDtypeStruct((B, G, P), f32),
            jax.ShapeDtypeStruct((B, G, P), f32),
        ],
    )(xx, xy, xz)

    cm = jnp.stack([cx, cy, cz], axis=-1)

    def b2(v):
        return v.reshape(1, -1)

    wspecs = [pl.BlockSpec(w.shape, lambda b, n=w.ndim: (0,) * n) for w in
              [We1, b2(be1), We2, b2(be2), Wp1, b2(bp1), Wp2, b2(bp2),
               Wq, Wk, Wv, Wo, Wf1, b2(bf1), Wf2, b2(bf2), Wout, b2(bout)]]

    out = pl.pallas_call(
        _encode_kernel,
        grid=(B,),
        in_specs=[
            pl.BlockSpec((1, G, P), lambda b: (b, 0, 0)),
            pl.BlockSpec((1, G, P), lambda b: (b, 0, 0)),
            pl.BlockSpec((1, G, P), lambda b: (b, 0, 0)),
            pl.BlockSpec((1, G, D), lambda b: (b, 0, 0)),
        ] + wspecs,
        out_specs=pl.BlockSpec((1, 1, OUT), lambda b: (b, 0, 0)),
        out_shape=jax.ShapeDtypeStruct((B, 1, OUT), f32),
    )(px, py, pz, cm,
      We1, b2(be1), We2, b2(be2), Wp1, b2(bp1), Wp2, b2(bp2),
      Wq, Wk, Wv, Wo, Wf1, b2(bf1), Wf2, b2(bf2), Wout, b2(bout))
    return out.reshape(B, OUT)
